# baseline (speedup 1.0000x reference)
import functools

import jax
import jax.numpy as jnp
import numpy as np
from jax import lax
from jax.experimental import pallas as pl
from jax.experimental.pallas import tpu as pltpu
from jax.experimental.pallas import tpu_sc as plsc

_N, _E, _D = 10000, 2000, 128
_TN = 400
_NT = _N // _TN
_BIG = np.float32(9e15)

_SC_NC, _SC_NS = 2, 16
_NW = _SC_NC * _SC_NS
_EPAD = 2048
_BPW = _EPAD // _NW


def _sc_root_gather(table, idx):
    mesh = plsc.VectorSubcoreMesh(core_axis_name="c", subcore_axis_name="s")

    @functools.partial(
        pl.kernel,
        mesh=mesh,
        out_type=jax.ShapeDtypeStruct((_EPAD, _D), jnp.float32),
        scratch_types=[
            pltpu.VMEM((_BPW,), jnp.int32),
            pltpu.VMEM((_BPW, _D), jnp.float32),
            pltpu.SemaphoreType.DMA,
        ],
    )
    def k(table_hbm, idx_hbm, out_hbm, idx_v, rows_v, sem):
        wid = lax.axis_index("s") * _SC_NC + lax.axis_index("c")
        base = wid * _BPW
        pltpu.sync_copy(idx_hbm.at[pl.ds(base, _BPW)], idx_v)
        pltpu.async_copy(table_hbm.at[idx_v], rows_v, sem).wait()
        pltpu.sync_copy(rows_v, out_hbm.at[pl.ds(base, _BPW)])

    return k(table, idx)


def _tc_body(adj_ref, x_ref, root_ref, W2_ref, W3_ref, bns_ref, bnb_ref,
             fw1_ref, fb1_ref, fw2_ref, fb2_ref,
             xout_ref, eout_ref,
             mask_s, aTx_s, deg_s, e4a_s, eaccT_s, cmax_s, csum_s, er_s):
    i = pl.program_id(0)
    s = i % _NT

    @pl.when(i < _NT)
    def _phase0():
        adj_t = adj_ref[...]
        x_t = x_ref[...]

        @pl.when(i == 0)
        def _init0():
            aTx_s[...] = jnp.zeros_like(aTx_s)
            deg_s[...] = jnp.zeros_like(deg_s)

        deg_s[...] += jnp.sum(adj_t, axis=0, keepdims=True)
        aTx_s[...] += lax.dot_general(adj_t, x_t, (((0,), (0,)), ((), ())))
        mask_s[s] = (adj_t > 0).astype(jnp.int8)

    @pl.when((i >= _NT) & (i < 2 * _NT))
    def _phase1():
        @pl.when(i == _NT)
        def _init1():
            degc = jnp.transpose(deg_s[...])
            edge0 = aTx_s[...] / (degc + 1e-10) + root_ref[...]
            e4a_s[...] = jnp.dot(edge0, W3_ref[...])
            eaccT_s[...] = jnp.zeros_like(eaccT_s)
            cmax_s[...] = jnp.full_like(cmax_s, -jnp.inf)
            csum_s[...] = jnp.zeros_like(csum_s)

        x_t = x_ref[...]
        x4a = jnp.dot(x_t, W2_ref[...])
        att = lax.dot_general(x4a, e4a_s[...],
                              (((1,), (1,)), ((), ())))
        att = att + (mask_s[s].astype(jnp.float32) - 1.0) * _BIG
        tmax = jnp.max(att, axis=0, keepdims=True)
        ncmax = jnp.maximum(cmax_s[...], tmax)
        scale = jnp.exp(cmax_s[...] - ncmax)
        p = jnp.exp(att - ncmax)
        csum_s[...] = csum_s[...] * scale + jnp.sum(p, axis=0, keepdims=True)
        eaccT_s[...] = eaccT_s[...] * scale + lax.dot_general(
            x_t, p, (((0,), (0,)), ((), ())))
        cmax_s[...] = ncmax

        @pl.when(i == 2 * _NT - 1)
        def _fin1():
            er = jnp.transpose(eaccT_s[...] / csum_s[...])
            er_s[...] = er
            e_elu = jnp.where(er > 0, er, jnp.exp(er) - 1.0)
            eout_ref[...] = e_elu * bns_ref[...] + bnb_ref[...]

    @pl.when(i >= 2 * _NT)
    def _phase2():
        x_t = x_ref[...]
        x4a = jnp.dot(x_t, W2_ref[...])
        att = lax.dot_general(x4a, e4a_s[...],
                              (((1,), (1,)), ((), ())))
        att = att + (mask_s[s].astype(jnp.float32) - 1.0) * _BIG
        rmax = jnp.max(att, axis=1, keepdims=True)
        p = jnp.exp(att - rmax)
        rsum = jnp.sum(p, axis=1, keepdims=True)
        node = jnp.dot(p, er_s[...]) / rsum
        node = jnp.where(node > 0, node, jnp.exp(node) - 1.0)
        node = node * bns_ref[...] + bnb_ref[...]
        h0 = jnp.tanh(jnp.dot(x_t, fw1_ref[...]) + fb1_ref[...])
        s0 = jnp.sum(h0 * fw2_ref[...], axis=1, keepdims=True) + fb2_ref[...]
        h1 = jnp.tanh(jnp.dot(node, fw1_ref[...]) + fb1_ref[...])
        s1 = jnp.sum(h1 * fw2_ref[...], axis=1, keepdims=True) + fb2_ref[...]
        mx = jnp.maximum(s0, s1)
        e0 = jnp.exp(s0 - mx)
        e1 = jnp.exp(s1 - mx)
        xout_ref[...] = (e0 * x_t + e1 * node) / (e0 + e1)


_TC_IN_SPECS = [
    pl.BlockSpec((_TN, _E), lambda i: (jnp.minimum(i, _NT - 1), 0)),
    pl.BlockSpec((_TN, _D), lambda i: (i % _NT, 0)),
    pl.BlockSpec((_E, _D), lambda i: (0, 0)),
    pl.BlockSpec((_D, _D), lambda i: (0, 0)),
    pl.BlockSpec((_D, _D), lambda i: (0, 0)),
    pl.BlockSpec((1, _D), lambda i: (0, 0)),
    pl.BlockSpec((1, _D), lambda i: (0, 0)),
    pl.BlockSpec((_D, _D), lambda i: (0, 0)),
    pl.BlockSpec((1, _D), lambda i: (0, 0)),
    pl.BlockSpec((1, _D), lambda i: (0, 0)),
    pl.BlockSpec((1, 1), lambda i: (0, 0)),
]

_TC_OUT_SPECS = [
    pl.BlockSpec((_TN, _D),
                 lambda i: (jnp.where(i < 2 * _NT, 0, i - 2 * _NT), 0)),
    pl.BlockSpec((_E, _D), lambda i: (0, 0)),
]

_TC_OUT_SHAPE = [
    jax.ShapeDtypeStruct((_N, _D), jnp.float32),
    jax.ShapeDtypeStruct((_E, _D), jnp.float32),
]

_TC_SCRATCH = [
    pltpu.VMEM((_NT, _TN, _E), jnp.int8),
    pltpu.VMEM((_E, _D), jnp.float32),
    pltpu.VMEM((1, _E), jnp.float32),
    pltpu.VMEM((_E, _D), jnp.float32),
    pltpu.VMEM((_D, _E), jnp.float32),
    pltpu.VMEM((1, _E), jnp.float32),
    pltpu.VMEM((1, _E), jnp.float32),
    pltpu.VMEM((_E, _D), jnp.float32),
]


def kernel(x, adj, root_index, W2, W3, bn_gamma, bn_beta, bn_mean, bn_var,
           fw1, fb1, fw2, fb2):
    idx = jnp.concatenate([root_index.astype(jnp.int32),
                           jnp.zeros((_EPAD - _E,), jnp.int32)])
    root_emb = _sc_root_gather(x, idx)[:_E]

    bn_scale = bn_gamma * lax.rsqrt(bn_var + 1e-5)
    bn_shift = bn_beta - bn_mean * bn_scale

    x_out, edge_out = pl.pallas_call(
        _tc_body,
        grid=(3 * _NT,),
        in_specs=_TC_IN_SPECS,
        out_specs=_TC_OUT_SPECS,
        out_shape=_TC_OUT_SHAPE,
        scratch_shapes=_TC_SCRATCH,
        compiler_params=pltpu.CompilerParams(
            dimension_semantics=("arbitrary",),
            vmem_limit_bytes=64 * 1024 * 1024,
        ),
    )(adj, x, root_emb, W2, W3, bn_scale.reshape(1, _D),
      bn_shift.reshape(1, _D), fw1, fb1.reshape(1, _D), fw2.reshape(1, _D),
      fb2.reshape(1, 1))
    return x_out, edge_out


# speedup vs baseline: 1.2822x; 1.2822x over previous
import functools

import jax
import jax.numpy as jnp
import numpy as np
from jax import lax
from jax.experimental import pallas as pl
from jax.experimental.pallas import tpu as pltpu
from jax.experimental.pallas import tpu_sc as plsc

_N, _E, _D = 10000, 2000, 128
_TN = 1000
_NT = _N // _TN
_BF = jnp.bfloat16
_F32 = jnp.float32

_SC_NC, _SC_NS = 2, 16
_NW = _SC_NC * _SC_NS
_EPAD = 2048
_BPW = _EPAD // _NW


def _sc_root_gather(table, idx):
    mesh = plsc.VectorSubcoreMesh(core_axis_name="c", subcore_axis_name="s")

    @functools.partial(
        pl.kernel,
        mesh=mesh,
        out_type=jax.ShapeDtypeStruct((_EPAD, _D), jnp.float32),
        scratch_types=[
            pltpu.VMEM((_BPW,), jnp.int32),
            pltpu.VMEM((_BPW, _D), jnp.float32),
            pltpu.SemaphoreType.DMA,
        ],
    )
    def k(table_hbm, idx_hbm, out_hbm, idx_v, rows_v, sem):
        wid = lax.axis_index("s") * _SC_NC + lax.axis_index("c")
        base = wid * _BPW
        pltpu.sync_copy(idx_hbm.at[pl.ds(base, _BPW)], idx_v)
        pltpu.async_copy(table_hbm.at[idx_v], rows_v, sem).wait()
        pltpu.sync_copy(rows_v, out_hbm.at[pl.ds(base, _BPW)])

    return k(table, idx)


def _nn(a, b):
    return lax.dot_general(a, b, (((1,), (0,)), ((), ())),
                           preferred_element_type=_F32)


def _tn(a, b):
    return lax.dot_general(a, b, (((0,), (0,)), ((), ())),
                           preferred_element_type=_F32)


def _nt(a, b):
    return lax.dot_general(a, b, (((1,), (1,)), ((), ())),
                           preferred_element_type=_F32)


def _tc_body(adj_ref, x_ref, root_ref, W2_ref, W3_ref, bns_ref, bnb_ref,
             fw1_ref, fb1_ref, fw2_ref, fb2_ref,
             xout_ref, eout_ref,
             aTx_s, deg_s, e4a_s, eaccT_s, csum_s, er_s):
    i = pl.program_id(0)
    ones_r = jnp.ones((1, _TN), _BF)

    @pl.when(i < _NT)
    def _phase0():
        adj_b = adj_ref[...].astype(_BF)
        x_b = x_ref[...].astype(_BF)

        @pl.when(i == 0)
        def _init0():
            aTx_s[...] = jnp.zeros_like(aTx_s)
            deg_s[...] = jnp.zeros_like(deg_s)

        deg_s[...] += _tn(ones_r.reshape(_TN, 1), adj_b)
        aTx_s[...] += _tn(adj_b, x_b)

    @pl.when((i >= _NT) & (i < 2 * _NT))
    def _phase1():
        @pl.when(i == _NT)
        def _init1():
            degc = jnp.transpose(deg_s[...])
            edge0 = aTx_s[...] / (degc + 1e-10) + root_ref[...]
            e4a_s[...] = _nn(edge0, W3_ref[...])
            eaccT_s[...] = jnp.zeros_like(eaccT_s)
            csum_s[...] = jnp.zeros_like(csum_s)

        x_t = x_ref[...]
        x_b = x_t.astype(_BF)
        x4a = _nn(x_t, W2_ref[...])
        att = _nt(x4a, e4a_s[...])
        pm = (jnp.exp(att) * adj_ref[...]).astype(_BF)
        csum_s[...] += _tn(ones_r.reshape(_TN, 1), pm)
        eaccT_s[...] += _tn(x_b, pm)

        @pl.when(i == 2 * _NT - 1)
        def _fin1():
            er = jnp.transpose(eaccT_s[...] / csum_s[...])
            er_s[...] = er.astype(_BF)
            e_elu = jnp.where(er > 0, er, jnp.exp(er) - 1.0)
            eout_ref[...] = e_elu * bns_ref[...] + bnb_ref[...]

    @pl.when(i >= 2 * _NT)
    def _phase2():
        x_t = x_ref[...]
        x4a = _nn(x_t, W2_ref[...])
        att = _nt(x4a, e4a_s[...])
        pm = (jnp.exp(att) * adj_ref[...]).astype(_BF)
        rsum = _nn(pm, jnp.ones((_E, 1), _BF))
        node = _nn(pm, er_s[...]) / rsum
        node = jnp.where(node > 0, node, jnp.exp(node) - 1.0)
        node = node * bns_ref[...] + bnb_ref[...]
        h0 = jnp.tanh(jnp.dot(x_t, fw1_ref[...]) + fb1_ref[...])
        s0 = jnp.sum(h0 * fw2_ref[...], axis=1, keepdims=True) + fb2_ref[...]
        h1 = jnp.tanh(jnp.dot(node, fw1_ref[...]) + fb1_ref[...])
        s1 = jnp.sum(h1 * fw2_ref[...], axis=1, keepdims=True) + fb2_ref[...]
        mx = jnp.maximum(s0, s1)
        e0 = jnp.exp(s0 - mx)
        e1 = jnp.exp(s1 - mx)
        xout_ref[...] = (e0 * x_t + e1 * node) / (e0 + e1)


_TC_IN_SPECS = [
    pl.BlockSpec((_TN, _E), lambda i: (i % _NT, 0)),
    pl.BlockSpec((_TN, _D), lambda i: (i % _NT, 0)),
    pl.BlockSpec((_E, _D), lambda i: (0, 0)),
    pl.BlockSpec((_D, _D), lambda i: (0, 0)),
    pl.BlockSpec((_D, _D), lambda i: (0, 0)),
    pl.BlockSpec((1, _D), lambda i: (0, 0)),
    pl.BlockSpec((1, _D), lambda i: (0, 0)),
    pl.BlockSpec((_D, _D), lambda i: (0, 0)),
    pl.BlockSpec((1, _D), lambda i: (0, 0)),
    pl.BlockSpec((1, _D), lambda i: (0, 0)),
    pl.BlockSpec((1, 1), lambda i: (0, 0)),
]

_TC_OUT_SPECS = [
    pl.BlockSpec((_TN, _D),
                 lambda i: (jnp.where(i < 2 * _NT, 0, i - 2 * _NT), 0)),
    pl.BlockSpec((_E, _D), lambda i: (0, 0)),
]

_TC_OUT_SHAPE = [
    jax.ShapeDtypeStruct((_N, _D), jnp.float32),
    jax.ShapeDtypeStruct((_E, _D), jnp.float32),
]

_TC_SCRATCH = [
    pltpu.VMEM((_E, _D), jnp.float32),
    pltpu.VMEM((1, _E), jnp.float32),
    pltpu.VMEM((_E, _D), jnp.float32),
    pltpu.VMEM((_D, _E), jnp.float32),
    pltpu.VMEM((1, _E), jnp.float32),
    pltpu.VMEM((_E, _D), _BF),
]


def kernel(x, adj, root_index, W2, W3, bn_gamma, bn_beta, bn_mean, bn_var,
           fw1, fb1, fw2, fb2):
    idx = jnp.concatenate([root_index.astype(jnp.int32),
                           jnp.zeros((_EPAD - _E,), jnp.int32)])
    root_emb = _sc_root_gather(x, idx)[:_E]

    bn_scale = bn_gamma * lax.rsqrt(bn_var + 1e-5)
    bn_shift = bn_beta - bn_mean * bn_scale

    x_out, edge_out = pl.pallas_call(
        _tc_body,
        grid=(3 * _NT,),
        in_specs=_TC_IN_SPECS,
        out_specs=_TC_OUT_SPECS,
        out_shape=_TC_OUT_SHAPE,
        scratch_shapes=_TC_SCRATCH,
        compiler_params=pltpu.CompilerParams(
            dimension_semantics=("arbitrary",),
            vmem_limit_bytes=64 * 1024 * 1024,
        ),
    )(adj, x, root_emb, W2, W3,
      bn_scale.reshape(1, _D), bn_shift.reshape(1, _D), fw1,
      fb1.reshape(1, _D), fw2.reshape(1, _D), fb2.reshape(1, 1))
    return x_out, edge_out
